# Initial kernel scaffold; baseline (speedup 1.0000x reference)
import functools

import jax
import jax.numpy as jnp
from jax import lax
from jax.experimental import pallas as pl
from jax.experimental.pallas import tpu as pltpu
from jax.experimental.pallas import tpu_sc as plsc

NC = 2
NS = 16
NW = NC * NS
LANES = 16
CHUNK = 128

_MESH = plsc.VectorSubcoreMesh(core_axis_name="c", subcore_axis_name="s",
                               num_cores=NC, num_subcores=NS)


def _fill(ref, base, n, value, dtype):
  def bd(i, _):
    ref[pl.ds(base + i * LANES, LANES)] = jnp.full((LANES,), value, dtype)
    return 0
  lax.fori_loop(0, n // LANES, bd, 0)


def _rsqrt_newton(x):
  i = plsc.bitcast(x, jnp.int32)
  i = jnp.full(i.shape, 0x5F3759DF, jnp.int32) - lax.shift_right_arithmetic(
      i, jnp.full(i.shape, 1, jnp.int32))
  y = plsc.bitcast(i, jnp.float32)
  for _ in range(3):
    y = y * (1.5 - 0.5 * x * y * y)
  return y


def _deg_dis_g0(npad, ncha, x_pad, col_a):
  rows_w = npad // NW
  stripe = npad // NS

  def body(cola_hbm, x_hbm, dis_hbm, g0_hbm,
           colv, onesv, zv, degv, disv, xv, deg_sh):
    c = lax.axis_index("c")
    s = lax.axis_index("s")
    w = c * NS + s

    _fill(onesv, 0, CHUNK, 1.0, jnp.float32)
    _fill(zv, 0, stripe, 0.0, jnp.float32)
    pltpu.sync_copy(zv, deg_sh.at[pl.ds(s * stripe, stripe)])
    plsc.subcore_barrier()

    pltpu.sync_copy(cola_hbm.at[s], colv)

    def ed(j, _):
      pltpu.sync_copy(onesv, deg_sh.at[colv.at[j]], add=True)
      return 0
    lax.fori_loop(0, ncha, ed, 0)
    plsc.subcore_barrier()

    base = w * rows_w
    pltpu.sync_copy(deg_sh.at[pl.ds(base, rows_w)], degv)

    def nd(i, _):
      dg = degv[pl.ds(i * LANES, LANES)] + 1.0
      disv[pl.ds(i * LANES, LANES)] = _rsqrt_newton(dg)
      return 0
    lax.fori_loop(0, rows_w // LANES, nd, 0)
    pltpu.sync_copy(disv, dis_hbm.at[pl.ds(base, rows_w)])

    pltpu.sync_copy(x_hbm.at[pl.ds(base, rows_w)], xv)

    def sc(i, _):
      sval = disv[i]
      for jj in range(8):
        xv[i, pl.ds(jj * LANES, LANES)] = xv[i, pl.ds(jj * LANES, LANES)] * sval
      return 0
    lax.fori_loop(0, rows_w, sc, 0)
    pltpu.sync_copy(xv, g0_hbm.at[pl.ds(base, rows_w)])

  f = pl.kernel(
      body,
      out_type=(jax.ShapeDtypeStruct((npad,), jnp.float32),
                jax.ShapeDtypeStruct((npad, 128), jnp.float32)),
      mesh=_MESH,
      scratch_types=[
          pltpu.VMEM((ncha, CHUNK), jnp.int32),
          pltpu.VMEM((CHUNK,), jnp.float32),
          pltpu.VMEM((npad // NS,), jnp.float32),
          pltpu.VMEM((rows_w,), jnp.float32),
          pltpu.VMEM((rows_w,), jnp.float32),
          pltpu.VMEM((rows_w, 128), jnp.float32),
          pltpu.VMEM_SHARED((npad,), jnp.float32),
      ],
  )
  return f(col_a, x_pad)


def _hop(npad, nch, g, row3, col3):
  stripe = npad // NS
  zrows = 128

  def body(g_hbm, row_hbm, col_hbm, part_hbm, rowv, colv, buf, zb, acc_sh):
    c = lax.axis_index("c")
    s = lax.axis_index("s")
    w = c * NS + s

    def zz(i, _):
      for jj in range(8):
        zb[i, pl.ds(jj * LANES, LANES)] = jnp.zeros((LANES,), jnp.float32)
      return 0
    lax.fori_loop(0, zrows, zz, 0)
    for k in range(stripe // zrows):
      pltpu.sync_copy(zb, acc_sh.at[pl.ds(s * stripe + k * zrows, zrows)])
    plsc.subcore_barrier()

    pltpu.sync_copy(row_hbm.at[w], rowv)
    pltpu.sync_copy(col_hbm.at[w], colv)

    def ed(j, _):
      pltpu.sync_copy(g_hbm.at[rowv.at[j]], buf)
      pltpu.sync_copy(buf, acc_sh.at[colv.at[j]], add=True)
      return 0
    lax.fori_loop(0, nch, ed, 0)
    plsc.subcore_barrier()

    pltpu.sync_copy(acc_sh.at[pl.ds(s * stripe, stripe)],
                    part_hbm.at[c, pl.ds(s * stripe, stripe)])

  f = pl.kernel(
      body,
      out_type=jax.ShapeDtypeStruct((NC, npad, 128), jnp.float32),
      mesh=_MESH,
      scratch_types=[
          pltpu.VMEM((nch, CHUNK), jnp.int32),
          pltpu.VMEM((nch, CHUNK), jnp.int32),
          pltpu.VMEM((CHUNK, 128), jnp.float32),
          pltpu.VMEM((zrows, 128), jnp.float32),
          pltpu.VMEM_SHARED((npad, 128), jnp.float32),
      ],
  )
  return f(g, row3, col3)


def _combine_kernel(dis_ref, p0_ref, p1_ref, g_ref, out_ref):
  dd = dis_ref[...]
  out_ref[...] = (dd * dd) * (p0_ref[...] + p1_ref[...] + g_ref[...])


def _final_kernel(nclass, dis_ref, q0_ref, q1_ref, g_ref, wt_ref, b_ref,
                  out_ref):
  h = dis_ref[...] * (q0_ref[...] + q1_ref[...] + g_ref[...])
  logits = jnp.dot(h, wt_ref[...], preferred_element_type=jnp.float32)
  logits = logits + b_ref[...]
  colid = lax.broadcasted_iota(jnp.int32, logits.shape, 1)
  valid = colid < nclass
  neg = jnp.where(valid, logits, -jnp.inf)
  m = jnp.max(neg, axis=1, keepdims=True)
  se = jnp.sum(jnp.where(valid, jnp.exp(logits - m), 0.0), axis=1,
               keepdims=True)
  out_ref[...] = logits - m - jnp.log(se)


def kernel(x, edge_index, W, b):
  n, d = x.shape
  e = edge_index.shape[1]
  nclass = W.shape[0]
  npad = ((n + 255) // 256) * 256
  if npad - n < 8:
    npad += 256

  ei = edge_index.astype(jnp.int32)
  row = ei[0]
  col = ei[1]

  per_w = CHUNK * ((e + NW * CHUNK - 1) // (NW * CHUNK))
  nch = per_w // CHUNK
  epad = NW * per_w
  pad = jnp.arange(epad - e, dtype=jnp.int32) % (npad - n) + n
  row3 = jnp.concatenate([row, pad]).reshape(NW, nch, CHUNK)
  col3 = jnp.concatenate([col, pad]).reshape(NW, nch, CHUNK)

  per_t = CHUNK * ((e + NS * CHUNK - 1) // (NS * CHUNK))
  ncha = per_t // CHUNK
  epad_a = NS * per_t
  pad_a = jnp.arange(epad_a - e, dtype=jnp.int32) % (npad - n) + n
  col_a = jnp.concatenate([col, pad_a]).reshape(NS, ncha, CHUNK)

  x_pad = jnp.zeros((npad, d), jnp.float32).at[:n].set(x)

  dis, g0 = _deg_dis_g0(npad, ncha, x_pad, col_a)
  part1 = _hop(npad, nch, g0, row3, col3)
  dis2d = dis.reshape(npad, 1)

  rb = 1024
  grid = (npad // rb,) if npad % 1024 == 0 else (npad // 256,)
  rb = npad // grid[0]
  g1 = pl.pallas_call(
      _combine_kernel,
      grid=grid,
      in_specs=[
          pl.BlockSpec((rb, 1), lambda i: (i, 0)),
          pl.BlockSpec((rb, d), lambda i: (i, 0)),
          pl.BlockSpec((rb, d), lambda i: (i, 0)),
          pl.BlockSpec((rb, d), lambda i: (i, 0)),
      ],
      out_specs=pl.BlockSpec((rb, d), lambda i: (i, 0)),
      out_shape=jax.ShapeDtypeStruct((npad, d), jnp.float32),
  )(dis2d, part1[0], part1[1], g0)

  part2 = _hop(npad, nch, g1, row3, col3)

  wt = jnp.zeros((d, d), jnp.float32).at[:, :nclass].set(W.T.astype(jnp.float32))
  bp = jnp.zeros((1, d), jnp.float32).at[0, :nclass].set(b.astype(jnp.float32))
  out = pl.pallas_call(
      functools.partial(_final_kernel, nclass),
      grid=grid,
      in_specs=[
          pl.BlockSpec((rb, 1), lambda i: (i, 0)),
          pl.BlockSpec((rb, d), lambda i: (i, 0)),
          pl.BlockSpec((rb, d), lambda i: (i, 0)),
          pl.BlockSpec((rb, d), lambda i: (i, 0)),
          pl.BlockSpec((d, d), lambda i: (0, 0)),
          pl.BlockSpec((1, d), lambda i: (0, 0)),
      ],
      out_specs=pl.BlockSpec((rb, d), lambda i: (i, 0)),
      out_shape=jax.ShapeDtypeStruct((npad, d), jnp.float32),
  )(dis2d, part2[0], part2[1], g1, wt, bp)

  return out[:n, :nclass]


# speedup vs baseline: 21.9057x; 21.9057x over previous
import functools

import jax
import jax.numpy as jnp
from jax import lax
from jax.experimental import pallas as pl
from jax.experimental.pallas import tpu as pltpu
from jax.experimental.pallas import tpu_sc as plsc

NC = 2
NS = 16
NW = NC * NS
LANES = 16
CHUNK = 128

_MESH = plsc.VectorSubcoreMesh(core_axis_name="c", subcore_axis_name="s",
                               num_cores=NC, num_subcores=NS)


def _i32(v):
  return jnp.asarray(v, jnp.int32)


def _fill(ref, base, n, value, dtype):
  def bd(i, _):
    ref[pl.ds(_i32(base) + i * _i32(LANES), LANES)] = jnp.full(
        (LANES,), value, dtype)
    return _
  lax.fori_loop(_i32(0), _i32(n // LANES), bd, _i32(0))


def _rsqrt_newton(x):
  i = plsc.bitcast(x, jnp.int32)
  i = jnp.full(i.shape, 0x5F3759DF, jnp.int32) - lax.shift_right_arithmetic(
      i, jnp.full(i.shape, 1, jnp.int32))
  y = plsc.bitcast(i, jnp.float32)
  for _ in range(3):
    y = y * (1.5 - 0.5 * x * y * y)
  return y


def _deg_dis_g0(npad, ncha, x_pad, col_a):
  rows_w = npad // NW
  stripe = npad // NS

  def body(cola_hbm, x_hbm, dis_hbm, g0_hbm,
           colv, onesv, zv, degv, disv, xv, deg_sh):
    c = _i32(lax.axis_index("c"))
    s = _i32(lax.axis_index("s"))
    w = c * _i32(NS) + s

    _fill(onesv, 0, CHUNK, 1.0, jnp.float32)
    _fill(zv, 0, stripe, 0.0, jnp.float32)
    pltpu.sync_copy(zv, deg_sh.at[pl.ds(s * _i32(stripe), stripe)])
    plsc.subcore_barrier()

    pltpu.sync_copy(cola_hbm.at[s], colv)

    def ed(j, _):
      pltpu.sync_copy(onesv, deg_sh.at[colv.at[j]], add=True)
      return _
    lax.fori_loop(_i32(0), _i32(ncha), ed, _i32(0))
    plsc.subcore_barrier()

    base = w * _i32(rows_w)
    pltpu.sync_copy(deg_sh.at[pl.ds(base, rows_w)], degv)

    def nd(i, _):
      off = i * _i32(LANES)
      dg = degv[pl.ds(off, LANES)] + 1.0
      disv[pl.ds(off, LANES)] = _rsqrt_newton(dg)
      return _
    lax.fori_loop(_i32(0), _i32(rows_w // LANES), nd, _i32(0))
    pltpu.sync_copy(disv, dis_hbm.at[pl.ds(base, rows_w)])

    pltpu.sync_copy(x_hbm.at[pl.ds(base, rows_w)], xv)

    def sc(i, _):
      dd = disv[pl.ds(i * _i32(LANES), LANES)]
      for r in range(LANES):
        sval = dd[r]
        ri = i * _i32(LANES) + _i32(r)
        for jj in range(8):
          xv[ri, pl.ds(jj * LANES, LANES)] = (
              xv[ri, pl.ds(jj * LANES, LANES)] * sval)
      return _
    lax.fori_loop(_i32(0), _i32(rows_w // LANES), sc, _i32(0))
    pltpu.sync_copy(xv, g0_hbm.at[pl.ds(base, rows_w)])

  f = pl.kernel(
      body,
      out_type=(jax.ShapeDtypeStruct((npad,), jnp.float32),
                jax.ShapeDtypeStruct((npad, 128), jnp.float32)),
      mesh=_MESH,
      compiler_params=pltpu.CompilerParams(needs_layout_passes=False),
      scratch_types=[
          pltpu.VMEM((ncha, CHUNK), jnp.int32),
          pltpu.VMEM((CHUNK,), jnp.float32),
          pltpu.VMEM((npad // NS,), jnp.float32),
          pltpu.VMEM((rows_w,), jnp.float32),
          pltpu.VMEM((rows_w,), jnp.float32),
          pltpu.VMEM((rows_w, 128), jnp.float32),
          pltpu.VMEM_SHARED((npad,), jnp.float32),
      ],
  )
  return f(col_a, x_pad)


def _hop(npad, nch, g, row3, col3):
  stripe = npad // NS
  zrows = 128

  def body(g_hbm, row_hbm, col_hbm, part_hbm, rowv, colv, buf, acc_sh):
    c = _i32(lax.axis_index("c"))
    s = _i32(lax.axis_index("s"))
    w = c * _i32(NS) + s

    def zz(i, _):
      for jj in range(8):
        buf[i, pl.ds(jj * LANES, LANES)] = jnp.zeros((LANES,), jnp.float32)
      return _
    lax.fori_loop(_i32(0), _i32(zrows), zz, _i32(0))
    for k in range(stripe // zrows):
      pltpu.sync_copy(buf, acc_sh.at[pl.ds(s * _i32(stripe) + _i32(k * zrows), zrows)])
    plsc.subcore_barrier()

    pltpu.sync_copy(row_hbm.at[w], rowv)
    pltpu.sync_copy(col_hbm.at[w], colv)

    def ed(j, _):
      pltpu.sync_copy(g_hbm.at[rowv.at[j]], buf)
      pltpu.sync_copy(buf, acc_sh.at[colv.at[j]], add=True)
      return _
    lax.fori_loop(_i32(0), _i32(nch), ed, _i32(0))
    plsc.subcore_barrier()

    pltpu.sync_copy(acc_sh.at[pl.ds(s * _i32(stripe), stripe)],
                    part_hbm.at[c, pl.ds(s * _i32(stripe), stripe)])

  f = pl.kernel(
      body,
      out_type=jax.ShapeDtypeStruct((NC, npad, 128), jnp.float32),
      mesh=_MESH,
      compiler_params=pltpu.CompilerParams(needs_layout_passes=False),
      scratch_types=[
          pltpu.VMEM((nch, CHUNK), jnp.int32),
          pltpu.VMEM((nch, CHUNK), jnp.int32),
          pltpu.VMEM((CHUNK, 128), jnp.float32),
          pltpu.VMEM_SHARED((npad, 128), jnp.float32),
      ],
  )
  return f(g, row3, col3)


def _combine_kernel(dis_ref, p0_ref, p1_ref, g_ref, out_ref):
  dd = dis_ref[...]
  out_ref[...] = (dd * dd) * (p0_ref[...] + p1_ref[...] + g_ref[...])


def _final_kernel(nclass, dis_ref, q0_ref, q1_ref, g_ref, wt_ref, b_ref,
                  out_ref):
  h = dis_ref[...] * (q0_ref[...] + q1_ref[...] + g_ref[...])
  logits = jnp.dot(h, wt_ref[...], preferred_element_type=jnp.float32)
  logits = logits + b_ref[...]
  colid = lax.broadcasted_iota(jnp.int32, logits.shape, 1)
  valid = colid < nclass
  neg = jnp.where(valid, logits, -jnp.inf)
  m = jnp.max(neg, axis=1, keepdims=True)
  se = jnp.sum(jnp.where(valid, jnp.exp(logits - m), 0.0), axis=1,
               keepdims=True)
  out_ref[...] = logits - m - jnp.log(se)


def kernel(x, edge_index, W, b):
  n, d = x.shape
  e = edge_index.shape[1]
  nclass = W.shape[0]
  npad = ((n + 255) // 256) * 256
  if npad - n < 8:
    npad += 256

  ei = edge_index.astype(jnp.int32)
  row = ei[0]
  col = ei[1]

  per_w = CHUNK * ((e + NW * CHUNK - 1) // (NW * CHUNK))
  nch = per_w // CHUNK
  epad = NW * per_w
  pad = jnp.arange(epad - e, dtype=jnp.int32) % (npad - n) + n
  row3 = jnp.concatenate([row, pad]).reshape(NW, nch, CHUNK)
  col3 = jnp.concatenate([col, pad]).reshape(NW, nch, CHUNK)

  per_t = CHUNK * ((e + NS * CHUNK - 1) // (NS * CHUNK))
  ncha = per_t // CHUNK
  epad_a = NS * per_t
  pad_a = jnp.arange(epad_a - e, dtype=jnp.int32) % (npad - n) + n
  col_a = jnp.concatenate([col, pad_a]).reshape(NS, ncha, CHUNK)

  x_pad = jnp.zeros((npad, d), jnp.float32).at[:n].set(x)

  dis, g0 = _deg_dis_g0(npad, ncha, x_pad, col_a)
  part1 = _hop(npad, nch, g0, row3, col3)
  dis2d = dis.reshape(npad, 1)

  rb = 1024
  grid = (npad // rb,) if npad % 1024 == 0 else (npad // 256,)
  rb = npad // grid[0]
  g1 = pl.pallas_call(
      _combine_kernel,
      grid=grid,
      in_specs=[
          pl.BlockSpec((rb, 1), lambda i: (i, _i32(0))),
          pl.BlockSpec((rb, d), lambda i: (i, _i32(0))),
          pl.BlockSpec((rb, d), lambda i: (i, _i32(0))),
          pl.BlockSpec((rb, d), lambda i: (i, _i32(0))),
      ],
      out_specs=pl.BlockSpec((rb, d), lambda i: (i, _i32(0))),
      out_shape=jax.ShapeDtypeStruct((npad, d), jnp.float32),
  )(dis2d, part1[0], part1[1], g0)

  part2 = _hop(npad, nch, g1, row3, col3)

  wt = jnp.zeros((d, d), jnp.float32).at[:, :nclass].set(W.T.astype(jnp.float32))
  bp = jnp.zeros((1, d), jnp.float32).at[0, :nclass].set(b.astype(jnp.float32))
  out = pl.pallas_call(
      functools.partial(_final_kernel, nclass),
      grid=grid,
      in_specs=[
          pl.BlockSpec((rb, 1), lambda i: (i, _i32(0))),
          pl.BlockSpec((rb, d), lambda i: (i, _i32(0))),
          pl.BlockSpec((rb, d), lambda i: (i, _i32(0))),
          pl.BlockSpec((rb, d), lambda i: (i, _i32(0))),
          pl.BlockSpec((d, d), lambda i: (_i32(0), _i32(0))),
          pl.BlockSpec((1, d), lambda i: (_i32(0), _i32(0))),
      ],
      out_specs=pl.BlockSpec((rb, d), lambda i: (i, _i32(0))),
      out_shape=jax.ShapeDtypeStruct((npad, d), jnp.float32),
  )(dis2d, part2[0], part2[1], g1, wt, bp)

  return out[:n, :nclass]


# speedup vs baseline: 29.6504x; 1.3535x over previous
import functools

import jax
import jax.numpy as jnp
from jax import lax
from jax.experimental import pallas as pl
from jax.experimental.pallas import tpu as pltpu
from jax.experimental.pallas import tpu_sc as plsc

NC = 2
NS = 16
NW = NC * NS
LANES = 16
CHUNK = 128
WIN = 16

_MESH = plsc.VectorSubcoreMesh(core_axis_name="c", subcore_axis_name="s",
                               num_cores=NC, num_subcores=NS)


def _i32(v):
  return jnp.asarray(v, jnp.int32)


def _fill(ref, base, n, value, dtype):
  def bd(i, _):
    ref[pl.ds(_i32(base) + i * _i32(LANES), LANES)] = jnp.full(
        (LANES,), value, dtype)
    return _
  lax.fori_loop(_i32(0), _i32(n // LANES), bd, _i32(0))


def _rsqrt_newton(x):
  i = plsc.bitcast(x, jnp.int32)
  i = jnp.full(i.shape, 0x5F3759DF, jnp.int32) - lax.shift_right_arithmetic(
      i, jnp.full(i.shape, 1, jnp.int32))
  y = plsc.bitcast(i, jnp.float32)
  for _ in range(3):
    y = y * (1.5 - 0.5 * x * y * y)
  return y


def _deg_dis_g0(npad, ncha, x_pad, col_a):
  rows_w = npad // NW
  stripe = npad // NS

  def body(cola_hbm, x_hbm, dis_hbm, g0_hbm,
           colv, onesv, zv, degv, disv, xv, deg_sh):
    c = _i32(lax.axis_index("c"))
    s = _i32(lax.axis_index("s"))
    w = c * _i32(NS) + s

    _fill(onesv, 0, CHUNK, 1.0, jnp.float32)
    _fill(zv, 0, stripe, 0.0, jnp.float32)
    pltpu.sync_copy(zv, deg_sh.at[pl.ds(s * _i32(stripe), stripe)])
    plsc.subcore_barrier()

    pltpu.sync_copy(cola_hbm.at[s], colv)

    def ed(j, _):
      pltpu.sync_copy(onesv, deg_sh.at[colv.at[j]], add=True)
      return _
    lax.fori_loop(_i32(0), _i32(ncha), ed, _i32(0))
    plsc.subcore_barrier()

    base = w * _i32(rows_w)
    pltpu.sync_copy(deg_sh.at[pl.ds(base, rows_w)], degv)

    def nd(i, _):
      off = i * _i32(LANES)
      dg = degv[pl.ds(off, LANES)] + 1.0
      disv[pl.ds(off, LANES)] = _rsqrt_newton(dg)
      return _
    lax.fori_loop(_i32(0), _i32(rows_w // LANES), nd, _i32(0))
    pltpu.sync_copy(disv, dis_hbm.at[pl.ds(base, rows_w)])

    pltpu.sync_copy(x_hbm.at[pl.ds(base, rows_w)], xv)

    def sc(i, _):
      dd = disv[pl.ds(i * _i32(LANES), LANES)]
      for r in range(LANES):
        sval = dd[r]
        ri = i * _i32(LANES) + _i32(r)
        for jj in range(8):
          xv[ri, pl.ds(jj * LANES, LANES)] = (
              xv[ri, pl.ds(jj * LANES, LANES)] * sval)
      return _
    lax.fori_loop(_i32(0), _i32(rows_w // LANES), sc, _i32(0))
    pltpu.sync_copy(xv, g0_hbm.at[pl.ds(base, rows_w)])

  f = pl.kernel(
      body,
      out_type=(jax.ShapeDtypeStruct((npad,), jnp.float32),
                jax.ShapeDtypeStruct((npad, 128), jnp.float32)),
      mesh=_MESH,
      compiler_params=pltpu.CompilerParams(needs_layout_passes=False),
      scratch_types=[
          pltpu.VMEM((ncha, CHUNK), jnp.int32),
          pltpu.VMEM((CHUNK,), jnp.float32),
          pltpu.VMEM((npad // NS,), jnp.float32),
          pltpu.VMEM((rows_w,), jnp.float32),
          pltpu.VMEM((rows_w,), jnp.float32),
          pltpu.VMEM((rows_w, 128), jnp.float32),
          pltpu.VMEM_SHARED((npad,), jnp.float32),
      ],
  )
  return f(col_a, x_pad)


def _hop(npad, nch, g, row3, col3):
  stripe = npad // NS
  zrows = 128
  nblk = nch // WIN

  def body(g_hbm, row_hbm, col_hbm, part_hbm, rowin, colin, buf0, buf1,
           acc_sh, sem):
    c = _i32(lax.axis_index("c"))
    s = _i32(lax.axis_index("s"))
    w = c * _i32(NS) + s

    def zz(i, _):
      for jj in range(8):
        buf0[i, pl.ds(jj * LANES, LANES)] = jnp.zeros((LANES,), jnp.float32)
      return _
    lax.fori_loop(_i32(0), _i32(zrows), zz, _i32(0))
    for k in range(stripe // zrows):
      pltpu.sync_copy(buf0, acc_sh.at[pl.ds(s * _i32(stripe) + _i32(k * zrows), zrows)])
    plsc.subcore_barrier()

    def blk(b, _):
      base = b * _i32(WIN)
      pltpu.sync_copy(row_hbm.at[w, pl.ds(base, WIN)], rowin)
      pltpu.sync_copy(col_hbm.at[w, pl.ds(base, WIN)], colin)
      pltpu.async_copy(g_hbm.at[rowin.at[_i32(0)]], buf0, sem)

      def ed(i, _):
        j = _i32(2) * i
        pltpu.async_copy(g_hbm.at[rowin.at[j + _i32(1)]], buf1, sem)
        pltpu.make_async_copy(g_hbm.at[rowin.at[_i32(0)]], buf0, sem).wait()
        pltpu.sync_copy(buf0, acc_sh.at[colin.at[j]], add=True)
        pltpu.async_copy(g_hbm.at[rowin.at[j + _i32(2)]], buf0, sem)
        pltpu.make_async_copy(g_hbm.at[rowin.at[_i32(0)]], buf1, sem).wait()
        pltpu.sync_copy(buf1, acc_sh.at[colin.at[j + _i32(1)]], add=True)
        return _
      lax.fori_loop(_i32(0), _i32(WIN // 2 - 1), ed, _i32(0))

      jl = _i32(WIN - 2)
      pltpu.async_copy(g_hbm.at[rowin.at[jl + _i32(1)]], buf1, sem)
      pltpu.make_async_copy(g_hbm.at[rowin.at[_i32(0)]], buf0, sem).wait()
      pltpu.sync_copy(buf0, acc_sh.at[colin.at[jl]], add=True)
      pltpu.make_async_copy(g_hbm.at[rowin.at[_i32(0)]], buf1, sem).wait()
      pltpu.sync_copy(buf1, acc_sh.at[colin.at[jl + _i32(1)]], add=True)
      return _
    lax.fori_loop(_i32(0), _i32(nblk), blk, _i32(0))
    plsc.subcore_barrier()

    pltpu.sync_copy(acc_sh.at[pl.ds(s * _i32(stripe), stripe)],
                    part_hbm.at[c, pl.ds(s * _i32(stripe), stripe)])

  f = pl.kernel(
      body,
      out_type=jax.ShapeDtypeStruct((NC, npad, 128), jnp.float32),
      mesh=_MESH,
      compiler_params=pltpu.CompilerParams(needs_layout_passes=False),
      scratch_types=[
          pltpu.VMEM((WIN, CHUNK), jnp.int32),
          pltpu.VMEM((WIN, CHUNK), jnp.int32),
          pltpu.VMEM((CHUNK, 128), jnp.float32),
          pltpu.VMEM((CHUNK, 128), jnp.float32),
          pltpu.VMEM_SHARED((npad, 128), jnp.float32),
          pltpu.SemaphoreType.DMA,
      ],
  )
  return f(g, row3, col3)


def _combine_kernel(dis_ref, p0_ref, p1_ref, g_ref, out_ref):
  dd = dis_ref[...]
  out_ref[...] = (dd * dd) * (p0_ref[...] + p1_ref[...] + g_ref[...])


def _final_kernel(nclass, dis_ref, q0_ref, q1_ref, g_ref, wt_ref, b_ref,
                  out_ref):
  h = dis_ref[...] * (q0_ref[...] + q1_ref[...] + g_ref[...])
  logits = jnp.dot(h, wt_ref[...], preferred_element_type=jnp.float32)
  logits = logits + b_ref[...]
  colid = lax.broadcasted_iota(jnp.int32, logits.shape, 1)
  valid = colid < nclass
  neg = jnp.where(valid, logits, -jnp.inf)
  m = jnp.max(neg, axis=1, keepdims=True)
  se = jnp.sum(jnp.where(valid, jnp.exp(logits - m), 0.0), axis=1,
               keepdims=True)
  out_ref[...] = logits - m - jnp.log(se)


def kernel(x, edge_index, W, b):
  n, d = x.shape
  e = edge_index.shape[1]
  nclass = W.shape[0]
  npad = ((n + 255) // 256) * 256
  if npad - n < 8:
    npad += 256

  ei = edge_index.astype(jnp.int32)
  row = ei[0]
  col = ei[1]

  nch = (e + NW * CHUNK - 1) // (NW * CHUNK)
  nch = ((nch + WIN - 1) // WIN) * WIN
  per_w = CHUNK * nch
  epad = NW * per_w
  pad = jnp.arange(epad - e, dtype=jnp.int32) % (npad - n) + n
  row3 = jnp.concatenate([row, pad]).reshape(NW, nch, CHUNK)
  col3 = jnp.concatenate([col, pad]).reshape(NW, nch, CHUNK)

  per_t = CHUNK * ((e + NS * CHUNK - 1) // (NS * CHUNK))
  ncha = per_t // CHUNK
  epad_a = NS * per_t
  pad_a = jnp.arange(epad_a - e, dtype=jnp.int32) % (npad - n) + n
  col_a = jnp.concatenate([col, pad_a]).reshape(NS, ncha, CHUNK)

  x_pad = jnp.zeros((npad, d), jnp.float32).at[:n].set(x)

  dis, g0 = _deg_dis_g0(npad, ncha, x_pad, col_a)
  part1 = _hop(npad, nch, g0, row3, col3)
  dis2d = dis.reshape(npad, 1)

  rb = 1024
  grid = (npad // rb,) if npad % 1024 == 0 else (npad // 256,)
  rb = npad // grid[0]
  g1 = pl.pallas_call(
      _combine_kernel,
      grid=grid,
      in_specs=[
          pl.BlockSpec((rb, 1), lambda i: (i, _i32(0))),
          pl.BlockSpec((rb, d), lambda i: (i, _i32(0))),
          pl.BlockSpec((rb, d), lambda i: (i, _i32(0))),
          pl.BlockSpec((rb, d), lambda i: (i, _i32(0))),
      ],
      out_specs=pl.BlockSpec((rb, d), lambda i: (i, _i32(0))),
      out_shape=jax.ShapeDtypeStruct((npad, d), jnp.float32),
  )(dis2d, part1[0], part1[1], g0)

  part2 = _hop(npad, nch, g1, row3, col3)

  wt = jnp.zeros((d, d), jnp.float32).at[:, :nclass].set(W.T.astype(jnp.float32))
  bp = jnp.zeros((1, d), jnp.float32).at[0, :nclass].set(b.astype(jnp.float32))
  out = pl.pallas_call(
      functools.partial(_final_kernel, nclass),
      grid=grid,
      in_specs=[
          pl.BlockSpec((rb, 1), lambda i: (i, _i32(0))),
          pl.BlockSpec((rb, d), lambda i: (i, _i32(0))),
          pl.BlockSpec((rb, d), lambda i: (i, _i32(0))),
          pl.BlockSpec((rb, d), lambda i: (i, _i32(0))),
          pl.BlockSpec((d, d), lambda i: (_i32(0), _i32(0))),
          pl.BlockSpec((1, d), lambda i: (_i32(0), _i32(0))),
      ],
      out_specs=pl.BlockSpec((rb, d), lambda i: (i, _i32(0))),
      out_shape=jax.ShapeDtypeStruct((npad, d), jnp.float32),
  )(dis2d, part2[0], part2[1], g1, wt, bp)

  return out[:n, :nclass]


# speedup vs baseline: 30.0840x; 1.0146x over previous
import functools

import jax
import jax.numpy as jnp
from jax import lax
from jax.experimental import pallas as pl
from jax.experimental.pallas import tpu as pltpu
from jax.experimental.pallas import tpu_sc as plsc

NC = 2
NS = 16
NW = NC * NS
LANES = 16
CHUNK = 128
WIN = 16

_MESH = plsc.VectorSubcoreMesh(core_axis_name="c", subcore_axis_name="s",
                               num_cores=NC, num_subcores=NS)


def _i32(v):
  return jnp.asarray(v, jnp.int32)


def _fill(ref, base, n, value, dtype):
  def bd(i, _):
    ref[pl.ds(_i32(base) + i * _i32(LANES), LANES)] = jnp.full(
        (LANES,), value, dtype)
    return _
  lax.fori_loop(_i32(0), _i32(n // LANES), bd, _i32(0))


def _rsqrt_newton(x):
  i = plsc.bitcast(x, jnp.int32)
  i = jnp.full(i.shape, 0x5F3759DF, jnp.int32) - lax.shift_right_arithmetic(
      i, jnp.full(i.shape, 1, jnp.int32))
  y = plsc.bitcast(i, jnp.float32)
  for _ in range(3):
    y = y * (1.5 - 0.5 * x * y * y)
  return y


def _deg_dis_g0(npad, per_t, x_pad, col_a):
  rows_w = npad // NW
  stripe = npad // NS

  def body(cola_hbm, x_hbm, dis_hbm, g0_hbm,
           colv, onesv, zv, degv, disv, xv, deg_sh):
    c = _i32(lax.axis_index("c"))
    s = _i32(lax.axis_index("s"))
    w = c * _i32(NS) + s

    _fill(onesv, 0, per_t, 1.0, jnp.float32)
    _fill(zv, 0, stripe, 0.0, jnp.float32)
    pltpu.sync_copy(zv, deg_sh.at[pl.ds(s * _i32(stripe), stripe)])
    plsc.subcore_barrier()

    pltpu.sync_copy(cola_hbm.at[s], colv)
    pltpu.sync_copy(onesv, deg_sh.at[colv], add=True)
    plsc.subcore_barrier()

    base = w * _i32(rows_w)
    pltpu.sync_copy(deg_sh.at[pl.ds(base, rows_w)], degv)

    def nd(i, _):
      off = i * _i32(LANES)
      dg = degv[pl.ds(off, LANES)] + 1.0
      disv[pl.ds(off, LANES)] = _rsqrt_newton(dg)
      return _
    lax.fori_loop(_i32(0), _i32(rows_w // LANES), nd, _i32(0))
    pltpu.sync_copy(disv, dis_hbm.at[pl.ds(base, rows_w)])

    pltpu.sync_copy(x_hbm.at[pl.ds(base, rows_w)], xv)

    def sc(i, _):
      dd = disv[pl.ds(i * _i32(LANES), LANES)]
      for r in range(LANES):
        sval = dd[r]
        ri = i * _i32(LANES) + _i32(r)
        for jj in range(8):
          xv[ri, pl.ds(jj * LANES, LANES)] = (
              xv[ri, pl.ds(jj * LANES, LANES)] * sval)
      return _
    lax.fori_loop(_i32(0), _i32(rows_w // LANES), sc, _i32(0))
    pltpu.sync_copy(xv, g0_hbm.at[pl.ds(base, rows_w)])

  f = pl.kernel(
      body,
      out_type=(jax.ShapeDtypeStruct((npad,), jnp.float32),
                jax.ShapeDtypeStruct((npad, 128), jnp.float32)),
      mesh=_MESH,
      compiler_params=pltpu.CompilerParams(needs_layout_passes=False),
      scratch_types=[
          pltpu.VMEM((per_t,), jnp.int32),
          pltpu.VMEM((per_t,), jnp.float32),
          pltpu.VMEM((npad // NS,), jnp.float32),
          pltpu.VMEM((rows_w,), jnp.float32),
          pltpu.VMEM((rows_w,), jnp.float32),
          pltpu.VMEM((rows_w, 128), jnp.float32),
          pltpu.VMEM_SHARED((npad,), jnp.float32),
      ],
  )
  return f(col_a, x_pad)


def _hop(npad, nch, g, row3, col3):
  stripe = npad // NS
  zrows = 128
  nblk = nch // WIN

  def body(g_hbm, row_hbm, col_hbm, part_hbm, rowin, colin, buf0, buf1,
           acc_sh, sem):
    c = _i32(lax.axis_index("c"))
    s = _i32(lax.axis_index("s"))
    w = c * _i32(NS) + s

    def zz(i, _):
      for jj in range(8):
        buf0[i, pl.ds(jj * LANES, LANES)] = jnp.zeros((LANES,), jnp.float32)
      return _
    lax.fori_loop(_i32(0), _i32(zrows), zz, _i32(0))
    for k in range(stripe // zrows):
      pltpu.sync_copy(buf0, acc_sh.at[pl.ds(s * _i32(stripe) + _i32(k * zrows), zrows)])
    plsc.subcore_barrier()

    def blk(b, _):
      base = b * _i32(WIN)
      pltpu.sync_copy(row_hbm.at[w, pl.ds(base, WIN)], rowin)
      pltpu.sync_copy(col_hbm.at[w, pl.ds(base, WIN)], colin)
      pltpu.async_copy(g_hbm.at[rowin.at[_i32(0)]], buf0, sem)

      def ed(i, _):
        j = _i32(2) * i
        pltpu.async_copy(g_hbm.at[rowin.at[j + _i32(1)]], buf1, sem)
        pltpu.make_async_copy(g_hbm.at[rowin.at[_i32(0)]], buf0, sem).wait()
        pltpu.sync_copy(buf0, acc_sh.at[colin.at[j]], add=True)
        pltpu.async_copy(g_hbm.at[rowin.at[j + _i32(2)]], buf0, sem)
        pltpu.make_async_copy(g_hbm.at[rowin.at[_i32(0)]], buf1, sem).wait()
        pltpu.sync_copy(buf1, acc_sh.at[colin.at[j + _i32(1)]], add=True)
        return _
      lax.fori_loop(_i32(0), _i32(WIN // 2 - 1), ed, _i32(0))

      jl = _i32(WIN - 2)
      pltpu.async_copy(g_hbm.at[rowin.at[jl + _i32(1)]], buf1, sem)
      pltpu.make_async_copy(g_hbm.at[rowin.at[_i32(0)]], buf0, sem).wait()
      pltpu.sync_copy(buf0, acc_sh.at[colin.at[jl]], add=True)
      pltpu.make_async_copy(g_hbm.at[rowin.at[_i32(0)]], buf1, sem).wait()
      pltpu.sync_copy(buf1, acc_sh.at[colin.at[jl + _i32(1)]], add=True)
      return _
    lax.fori_loop(_i32(0), _i32(nblk), blk, _i32(0))
    plsc.subcore_barrier()

    pltpu.sync_copy(acc_sh.at[pl.ds(s * _i32(stripe), stripe)],
                    part_hbm.at[c, pl.ds(s * _i32(stripe), stripe)])

  f = pl.kernel(
      body,
      out_type=jax.ShapeDtypeStruct((NC, npad, 128), jnp.float32),
      mesh=_MESH,
      compiler_params=pltpu.CompilerParams(needs_layout_passes=False),
      scratch_types=[
          pltpu.VMEM((WIN, CHUNK), jnp.int32),
          pltpu.VMEM((WIN, CHUNK), jnp.int32),
          pltpu.VMEM((CHUNK, 128), jnp.float32),
          pltpu.VMEM((CHUNK, 128), jnp.float32),
          pltpu.VMEM_SHARED((npad, 128), jnp.float32),
          pltpu.SemaphoreType.DMA,
      ],
  )
  return f(g, row3, col3)


def _combine_kernel(dis_ref, p0_ref, p1_ref, g_ref, out_ref):
  dd = dis_ref[...]
  out_ref[...] = (dd * dd) * (p0_ref[...] + p1_ref[...] + g_ref[...])


def _final_kernel(nclass, dis_ref, q0_ref, q1_ref, g_ref, wt_ref, b_ref,
                  out_ref):
  h = dis_ref[...] * (q0_ref[...] + q1_ref[...] + g_ref[...])
  logits = jnp.dot(h, wt_ref[...], preferred_element_type=jnp.float32)
  logits = logits + b_ref[...]
  colid = lax.broadcasted_iota(jnp.int32, logits.shape, 1)
  valid = colid < nclass
  neg = jnp.where(valid, logits, -jnp.inf)
  m = jnp.max(neg, axis=1, keepdims=True)
  se = jnp.sum(jnp.where(valid, jnp.exp(logits - m), 0.0), axis=1,
               keepdims=True)
  out_ref[...] = logits - m - jnp.log(se)


def kernel(x, edge_index, W, b):
  n, d = x.shape
  e = edge_index.shape[1]
  nclass = W.shape[0]
  npad = ((n + 255) // 256) * 256
  if npad - n < 8:
    npad += 256

  ei = edge_index.astype(jnp.int32)
  row = ei[0]
  col = ei[1]

  nch = (e + NW * CHUNK - 1) // (NW * CHUNK)
  nch = ((nch + WIN - 1) // WIN) * WIN
  per_w = CHUNK * nch
  epad = NW * per_w
  pad = jnp.arange(epad - e, dtype=jnp.int32) % (npad - n) + n
  row3 = jnp.concatenate([row, pad]).reshape(NW, nch, CHUNK)
  col3 = jnp.concatenate([col, pad]).reshape(NW, nch, CHUNK)

  per_t = CHUNK * ((e + NS * CHUNK - 1) // (NS * CHUNK))
  epad_a = NS * per_t
  pad_a = jnp.arange(epad_a - e, dtype=jnp.int32) % (npad - n) + n
  col_a = jnp.concatenate([col, pad_a]).reshape(NS, per_t)

  x_pad = jnp.zeros((npad, d), jnp.float32).at[:n].set(x)

  dis, g0 = _deg_dis_g0(npad, per_t, x_pad, col_a)
  part1 = _hop(npad, nch, g0, row3, col3)
  dis2d = dis.reshape(npad, 1)

  rb = 1024
  grid = (npad // rb,) if npad % 1024 == 0 else (npad // 256,)
  rb = npad // grid[0]
  g1 = pl.pallas_call(
      _combine_kernel,
      grid=grid,
      in_specs=[
          pl.BlockSpec((rb, 1), lambda i: (i, _i32(0))),
          pl.BlockSpec((rb, d), lambda i: (i, _i32(0))),
          pl.BlockSpec((rb, d), lambda i: (i, _i32(0))),
          pl.BlockSpec((rb, d), lambda i: (i, _i32(0))),
      ],
      out_specs=pl.BlockSpec((rb, d), lambda i: (i, _i32(0))),
      out_shape=jax.ShapeDtypeStruct((npad, d), jnp.float32),
  )(dis2d, part1[0], part1[1], g0)

  part2 = _hop(npad, nch, g1, row3, col3)

  wt = jnp.zeros((d, d), jnp.float32).at[:, :nclass].set(W.T.astype(jnp.float32))
  bp = jnp.zeros((1, d), jnp.float32).at[0, :nclass].set(b.astype(jnp.float32))
  out = pl.pallas_call(
      functools.partial(_final_kernel, nclass),
      grid=grid,
      in_specs=[
          pl.BlockSpec((rb, 1), lambda i: (i, _i32(0))),
          pl.BlockSpec((rb, d), lambda i: (i, _i32(0))),
          pl.BlockSpec((rb, d), lambda i: (i, _i32(0))),
          pl.BlockSpec((rb, d), lambda i: (i, _i32(0))),
          pl.BlockSpec((d, d), lambda i: (_i32(0), _i32(0))),
          pl.BlockSpec((1, d), lambda i: (_i32(0), _i32(0))),
      ],
      out_specs=pl.BlockSpec((rb, d), lambda i: (i, _i32(0))),
      out_shape=jax.ShapeDtypeStruct((npad, d), jnp.float32),
  )(dis2d, part2[0], part2[1], g1, wt, bp)

  return out[:n, :nclass]
